# baseline (speedup 1.0000x reference)
import functools

import jax
import jax.numpy as jnp
from jax import lax
from jax.experimental import pallas as pl
from jax.experimental.pallas import tpu as pltpu
from jax.experimental.pallas import tpu_sc as plsc

_NC = 2
_NS = 16
_NW = _NC * _NS
_K = 25
_S = 20
_NB = 8
_ZR = 16


def _make_agg(N, D, E):
    epw = E // _NW
    c = epw // _K
    nsc = c // _S
    rpt = (N // _NS) // 8 * 8
    tail = N - rpt * _NS
    mesh = plsc.VectorSubcoreMesh(core_axis_name="c", subcore_axis_name="s")

    @functools.partial(
        pl.kernel,
        out_type=jax.ShapeDtypeStruct((_NC, N, D), jnp.float32),
        mesh=mesh,
        scratch_types=[
            pltpu.VMEM((2, _S, _K), jnp.int32),
            pltpu.VMEM((2, _S, _K), jnp.int32),
            [pltpu.VMEM((_K, D), jnp.float32) for _ in range(_NB)],
            pltpu.VMEM((_ZR, D), jnp.float32),
            pltpu.VMEM_SHARED((N, D), jnp.float32),
            [pltpu.SemaphoreType.DMA for _ in range(_NB)],
            [pltpu.SemaphoreType.DMA for _ in range(_NB)],
            pltpu.SemaphoreType.DMA,
            pltpu.SemaphoreType.DMA,
            pltpu.SemaphoreType.DMA,
        ],
    )
    def agg(x_hbm, src_hbm, dst_hbm, out_hbm, src_v, dst_v, rows,
            zero_v, acc_sh, semg, sems, semi0, semi1, semz):
        cid = lax.axis_index("c")
        sid = lax.axis_index("s")
        wid = sid * _NC + cid

        def stage(s, buf, sem):
            pltpu.async_copy(src_hbm.at[wid, s], src_v.at[buf], sem)
            pltpu.async_copy(dst_hbm.at[wid, s], dst_v.at[buf], sem)

        def stage_wait(s, buf, sem):
            pltpu.make_async_copy(src_hbm.at[wid, s], src_v.at[buf],
                                  sem).wait()
            pltpu.make_async_copy(dst_hbm.at[wid, s], dst_v.at[buf],
                                  sem).wait()

        stage(0, 0, semi0)
        stage(1, 1, semi1)

        z16 = jnp.zeros((16,), jnp.float32)
        for i in range(_ZR):
            for j in range(D // 16):
                zero_v[i, pl.ds(j * 16, 16)] = z16

        nz = rpt // _ZR
        for t in range(nz):
            pltpu.async_copy(
                zero_v, acc_sh.at[pl.ds(sid * rpt + t * _ZR, _ZR)], semz)

        @pl.when(sid == _NS - 1)
        def _():
            pltpu.async_copy(zero_v.at[pl.ds(0, tail)],
                             acc_sh.at[pl.ds(_NS * rpt, tail)], semz)

        sb = 2 * _S

        def gfire(kk):
            buf, row = (kk // _S) % 2, kk % _S
            pltpu.async_copy(x_hbm.at[src_v.at[buf, row]], rows[kk % _NB],
                             semg[kk % _NB])

        def gwait(kk):
            buf, row = (kk // _S) % 2, kk % _S
            pltpu.make_async_copy(x_hbm.at[src_v.at[buf, row]],
                                  rows[kk % _NB], semg[kk % _NB]).wait()

        def sfire(kk):
            buf, row = (kk // _S) % 2, kk % _S
            pltpu.async_copy(rows[kk % _NB], acc_sh.at[dst_v.at[buf, row]],
                             sems[kk % _NB], add=True)

        def swait(kk):
            buf, row = (kk // _S) % 2, kk % _S
            pltpu.make_async_copy(rows[kk % _NB],
                                  acc_sh.at[dst_v.at[buf, row]],
                                  sems[kk % _NB]).wait()

        stage_wait(0, 0, semi0)
        for kk in range(_NB - 1):
            gfire(kk)

        nz = rpt // _ZR
        for t in range(nz):
            pltpu.make_async_copy(
                zero_v, acc_sh.at[pl.ds(sid * rpt + t * _ZR, _ZR)],
                semz).wait()

        @pl.when(sid == _NS - 1)
        def _():
            pltpu.make_async_copy(zero_v.at[pl.ds(0, tail)],
                                  acc_sh.at[pl.ds(_NS * rpt, tail)],
                                  semz).wait()

        plsc.subcore_barrier()

        def step(u, k, last_pair):
            if k == 0:
                if not last_pair:
                    @pl.when(u > 0)
                    def _():
                        swait(sb - 1)
                else:
                    swait(sb - 1)
            else:
                swait(k - 1)
            gwait(k)
            sfire(k)
            if k == 1 and not last_pair:
                @pl.when(u > 0)
                def _():
                    stage(2 * u + 1, 1, semi1)
            if k == 1 and last_pair:
                stage(nsc - 1, 1, semi1)
            if k == _S - (_NB - 1):
                stage_wait(2 * u + 1 if not last_pair else nsc - 1, 1, semi1)
            if k == _S + 1 and not last_pair:
                stage(2 * u + 2, 0, semi0)
            if k == sb - (_NB - 1) and not last_pair:
                stage_wait(2 * u + 2, 0, semi0)
            if k + _NB - 1 < sb or not last_pair:
                gfire(k + _NB - 1)

        def obody(u, carry):
            for k in range(sb):
                step(u, k, last_pair=False)
            return carry

        lax.fori_loop(0, nsc // 2 - 1, obody, 0)
        u_last = nsc // 2 - 1
        for k in range(sb):
            step(u_last, k, last_pair=True)
        swait(sb - 1)
        plsc.subcore_barrier()

        pltpu.sync_copy(acc_sh.at[pl.ds(sid * rpt, rpt)],
                        out_hbm.at[cid, pl.ds(sid * rpt, rpt)])

        @pl.when(sid == _NS - 1)
        def _():
            pltpu.sync_copy(acc_sh.at[pl.ds(_NS * rpt, tail)],
                            out_hbm.at[cid, pl.ds(_NS * rpt, tail)])

    return agg


def _make_mlp(N, D, H, O, blk, final):
    grid = (N // blk,)

    def body(eps_ref, x_ref, p_ref, w1_ref, b1_ref, w2_ref, b2_ref,
             *rest):
        if final:
            wf_ref, bf_ref, o_ref = rest
        else:
            (o_ref,) = rest
        h = x_ref[...] * (1.0 + eps_ref[0]) + p_ref[0] + p_ref[1]
        h = jnp.maximum(
            jnp.dot(h, w1_ref[...], preferred_element_type=jnp.float32)
            + b1_ref[...], 0.0)
        h = jnp.maximum(
            jnp.dot(h, w2_ref[...], preferred_element_type=jnp.float32)
            + b2_ref[...], 0.0)
        if final:
            h = jnp.dot(h, wf_ref[...],
                        preferred_element_type=jnp.float32) + bf_ref[...]
        o_ref[...] = h

    in_specs = [
        pl.BlockSpec(memory_space=pltpu.SMEM),
        pl.BlockSpec((blk, D), lambda i: (i, 0)),
        pl.BlockSpec((_NC, blk, D), lambda i: (0, i, 0)),
        pl.BlockSpec((D, H), lambda i: (0, 0)),
        pl.BlockSpec((1, H), lambda i: (0, 0)),
        pl.BlockSpec((H, H), lambda i: (0, 0)),
        pl.BlockSpec((1, H), lambda i: (0, 0)),
    ]
    if final:
        in_specs += [
            pl.BlockSpec((H, O), lambda i: (0, 0)),
            pl.BlockSpec((1, O), lambda i: (0, 0)),
        ]
    out_dim = O if final else H
    return pl.pallas_call(
        body,
        grid=grid,
        in_specs=in_specs,
        out_specs=pl.BlockSpec((blk, out_dim), lambda i: (i, 0)),
        out_shape=jax.ShapeDtypeStruct((N, out_dim), jnp.float32),
    )


def kernel(x, edge_index, eps, W1_0, b1_0, W2_0, b2_0, W1_1, b1_1, W2_1,
           b2_1, W1_2, b1_2, W2_2, b2_2, Wf, bf):
    N, D = x.shape
    E = edge_index.shape[1]
    H = W1_0.shape[1]
    O = Wf.shape[1]
    epw = E // _NW
    nsc = epw // _K // _S

    src3 = edge_index[0].reshape(_NW, nsc, _S, _K)
    dst3 = edge_index[1].reshape(_NW, nsc, _S, _K)

    agg = _make_agg(N, D, E)
    blk = 400
    mlp = _make_mlp(N, D, H, H, blk, final=False)
    mlp_final = _make_mlp(N, D, H, O, blk, final=True)

    layers = [(W1_0, b1_0, W2_0, b2_0), (W1_1, b1_1, W2_1, b2_1),
              (W1_2, b1_2, W2_2, b2_2)]
    h = x
    for l, (w1, b1, w2, b2) in enumerate(layers):
        p = agg(h, src3, dst3)
        eps_l = eps[l].reshape(1)
        args = (eps_l, h, p, w1, b1.reshape(1, -1), w2, b2.reshape(1, -1))
        if l == 2:
            h = mlp_final(*args, Wf, bf.reshape(1, -1))
        else:
            h = mlp(*args)
    return h


# speedup vs baseline: 1.0259x; 1.0259x over previous
import functools

import jax
import jax.numpy as jnp
from jax import lax
from jax.experimental import pallas as pl
from jax.experimental.pallas import tpu as pltpu
from jax.experimental.pallas import tpu_sc as plsc

_NC = 2
_NS = 16
_NW = _NC * _NS
_K = 50
_S = 20
_NB = 5
_ZR = 16


def _make_agg(N, D, E):
    epw = E // _NW
    c = epw // _K
    nsc = c // _S
    rpt = (N // _NS) // 8 * 8
    tail = N - rpt * _NS
    mesh = plsc.VectorSubcoreMesh(core_axis_name="c", subcore_axis_name="s")

    @functools.partial(
        pl.kernel,
        out_type=jax.ShapeDtypeStruct((_NC, N, D), jnp.float32),
        mesh=mesh,
        scratch_types=[
            pltpu.VMEM((2, _S, _K), jnp.int32),
            pltpu.VMEM((2, _S, _K), jnp.int32),
            [pltpu.VMEM((_K, D), jnp.float32) for _ in range(_NB)],
            pltpu.VMEM((_ZR, D), jnp.float32),
            pltpu.VMEM_SHARED((N, D), jnp.float32),
            [pltpu.SemaphoreType.DMA for _ in range(_NB)],
            [pltpu.SemaphoreType.DMA for _ in range(_NB)],
            pltpu.SemaphoreType.DMA,
            pltpu.SemaphoreType.DMA,
            pltpu.SemaphoreType.DMA,
        ],
    )
    def agg(x_hbm, src_hbm, dst_hbm, out_hbm, src_v, dst_v, rows,
            zero_v, acc_sh, semg, sems, semi0, semi1, semz):
        cid = lax.axis_index("c")
        sid = lax.axis_index("s")
        wid = sid * _NC + cid

        def stage(s, buf, sem):
            pltpu.async_copy(src_hbm.at[wid, s], src_v.at[buf], sem)
            pltpu.async_copy(dst_hbm.at[wid, s], dst_v.at[buf], sem)

        def stage_wait(s, buf, sem):
            pltpu.make_async_copy(src_hbm.at[wid, s], src_v.at[buf],
                                  sem).wait()
            pltpu.make_async_copy(dst_hbm.at[wid, s], dst_v.at[buf],
                                  sem).wait()

        stage(0, 0, semi0)
        stage(1, 1, semi1)

        z16 = jnp.zeros((16,), jnp.float32)
        for i in range(_ZR):
            for j in range(D // 16):
                zero_v[i, pl.ds(j * 16, 16)] = z16

        nz = rpt // _ZR
        for t in range(nz):
            pltpu.async_copy(
                zero_v, acc_sh.at[pl.ds(sid * rpt + t * _ZR, _ZR)], semz)

        @pl.when(sid == _NS - 1)
        def _():
            pltpu.async_copy(zero_v.at[pl.ds(0, tail)],
                             acc_sh.at[pl.ds(_NS * rpt, tail)], semz)

        sb = 2 * _S

        def gfire(kk):
            buf, row = (kk // _S) % 2, kk % _S
            pltpu.async_copy(x_hbm.at[src_v.at[buf, row]], rows[kk % _NB],
                             semg[kk % _NB])

        def gwait(kk):
            buf, row = (kk // _S) % 2, kk % _S
            pltpu.make_async_copy(x_hbm.at[src_v.at[buf, row]],
                                  rows[kk % _NB], semg[kk % _NB]).wait()

        def sfire(kk):
            buf, row = (kk // _S) % 2, kk % _S
            pltpu.async_copy(rows[kk % _NB], acc_sh.at[dst_v.at[buf, row]],
                             sems[kk % _NB], add=True)

        def swait(kk):
            buf, row = (kk // _S) % 2, kk % _S
            pltpu.make_async_copy(rows[kk % _NB],
                                  acc_sh.at[dst_v.at[buf, row]],
                                  sems[kk % _NB]).wait()

        stage_wait(0, 0, semi0)
        for kk in range(_NB - 1):
            gfire(kk)

        nz = rpt // _ZR
        for t in range(nz):
            pltpu.make_async_copy(
                zero_v, acc_sh.at[pl.ds(sid * rpt + t * _ZR, _ZR)],
                semz).wait()

        @pl.when(sid == _NS - 1)
        def _():
            pltpu.make_async_copy(zero_v.at[pl.ds(0, tail)],
                                  acc_sh.at[pl.ds(_NS * rpt, tail)],
                                  semz).wait()

        plsc.subcore_barrier()

        def step(u, k, last_pair):
            if k == 0:
                if not last_pair:
                    @pl.when(u > 0)
                    def _():
                        swait(sb - 1)
                else:
                    swait(sb - 1)
            else:
                swait(k - 1)
            gwait(k)
            sfire(k)
            if k == 1 and not last_pair:
                @pl.when(u > 0)
                def _():
                    stage(2 * u + 1, 1, semi1)
            if k == 1 and last_pair:
                stage(nsc - 1, 1, semi1)
            if k == _S - (_NB - 1):
                stage_wait(2 * u + 1 if not last_pair else nsc - 1, 1, semi1)
            if k == _S + 1 and not last_pair:
                stage(2 * u + 2, 0, semi0)
            if k == sb - (_NB - 1) and not last_pair:
                stage_wait(2 * u + 2, 0, semi0)
            if k + _NB - 1 < sb or not last_pair:
                gfire(k + _NB - 1)

        def obody(u, carry):
            for k in range(sb):
                step(u, k, last_pair=False)
            return carry

        lax.fori_loop(0, nsc // 2 - 1, obody, 0)
        u_last = nsc // 2 - 1
        for k in range(sb):
            step(u_last, k, last_pair=True)
        swait(sb - 1)
        plsc.subcore_barrier()

        pltpu.sync_copy(acc_sh.at[pl.ds(sid * rpt, rpt)],
                        out_hbm.at[cid, pl.ds(sid * rpt, rpt)])

        @pl.when(sid == _NS - 1)
        def _():
            pltpu.sync_copy(acc_sh.at[pl.ds(_NS * rpt, tail)],
                            out_hbm.at[cid, pl.ds(_NS * rpt, tail)])

    return agg


def _make_mlp(N, D, H, O, blk, final):
    grid = (N // blk,)

    def body(eps_ref, x_ref, p_ref, w1_ref, b1_ref, w2_ref, b2_ref,
             *rest):
        if final:
            wf_ref, bf_ref, o_ref = rest
        else:
            (o_ref,) = rest
        h = x_ref[...] * (1.0 + eps_ref[0]) + p_ref[0] + p_ref[1]
        h = jnp.maximum(
            jnp.dot(h, w1_ref[...], preferred_element_type=jnp.float32)
            + b1_ref[...], 0.0)
        h = jnp.maximum(
            jnp.dot(h, w2_ref[...], preferred_element_type=jnp.float32)
            + b2_ref[...], 0.0)
        if final:
            h = jnp.dot(h, wf_ref[...],
                        preferred_element_type=jnp.float32) + bf_ref[...]
        o_ref[...] = h

    in_specs = [
        pl.BlockSpec(memory_space=pltpu.SMEM),
        pl.BlockSpec((blk, D), lambda i: (i, 0)),
        pl.BlockSpec((_NC, blk, D), lambda i: (0, i, 0)),
        pl.BlockSpec((D, H), lambda i: (0, 0)),
        pl.BlockSpec((1, H), lambda i: (0, 0)),
        pl.BlockSpec((H, H), lambda i: (0, 0)),
        pl.BlockSpec((1, H), lambda i: (0, 0)),
    ]
    if final:
        in_specs += [
            pl.BlockSpec((H, O), lambda i: (0, 0)),
            pl.BlockSpec((1, O), lambda i: (0, 0)),
        ]
    out_dim = O if final else H
    return pl.pallas_call(
        body,
        grid=grid,
        in_specs=in_specs,
        out_specs=pl.BlockSpec((blk, out_dim), lambda i: (i, 0)),
        out_shape=jax.ShapeDtypeStruct((N, out_dim), jnp.float32),
    )


def kernel(x, edge_index, eps, W1_0, b1_0, W2_0, b2_0, W1_1, b1_1, W2_1,
           b2_1, W1_2, b1_2, W2_2, b2_2, Wf, bf):
    N, D = x.shape
    E = edge_index.shape[1]
    H = W1_0.shape[1]
    O = Wf.shape[1]
    epw = E // _NW
    nsc = epw // _K // _S

    src3 = edge_index[0].reshape(_NW, nsc, _S, _K)
    dst3 = edge_index[1].reshape(_NW, nsc, _S, _K)

    agg = _make_agg(N, D, E)
    blk = 400
    mlp = _make_mlp(N, D, H, H, blk, final=False)
    mlp_final = _make_mlp(N, D, H, O, blk, final=True)

    layers = [(W1_0, b1_0, W2_0, b2_0), (W1_1, b1_1, W2_1, b2_1),
              (W1_2, b1_2, W2_2, b2_2)]
    h = x
    for l, (w1, b1, w2, b2) in enumerate(layers):
        p = agg(h, src3, dst3)
        eps_l = eps[l].reshape(1)
        args = (eps_l, h, p, w1, b1.reshape(1, -1), w2, b2.reshape(1, -1))
        if l == 2:
            h = mlp_final(*args, Wf, bf.reshape(1, -1))
        else:
            h = mlp(*args)
    return h
